# baseline (speedup 1.0000x reference)
import functools

import jax
import jax.numpy as jnp
from jax import lax
from jax.experimental import pallas as pl
from jax.experimental.pallas import tpu as pltpu
from jax.experimental.pallas import tpu_sc as plsc

B, N, D, K = 16, 4096, 512, 2048
PD = 3
NC, NS = 2, 16
NW = NC * NS
WPG = NW // B
RPW = (B * K) // NW
CH = 32
NBUF = 6
NCH = RPW // CH
L = 16

_mesh = plsc.VectorSubcoreMesh(core_axis_name="c", subcore_axis_name="s")


@functools.partial(
    pl.kernel,
    out_type=(
        jax.ShapeDtypeStruct((B, K, D), jnp.float32),
        jax.ShapeDtypeStruct((B, PD, K), jnp.float32),
    ),
    mesh=_mesh,
    compiler_params=pltpu.CompilerParams(needs_layout_passes=False),
    scratch_types=[
        pltpu.VMEM((RPW,), jnp.int32),
        pltpu.VMEM((PD * N,), jnp.float32),
        pltpu.VMEM((NBUF, CH, D), jnp.float32),
        pltpu.VMEM((PD, RPW), jnp.float32),
        pltpu.SemaphoreType.DMA,
        pltpu.SemaphoreType.DMA,
        pltpu.SemaphoreType.DMA,
        pltpu.SemaphoreType.DMA,
    ],
)
def _node_drop_sc(x_hbm, post_hbm, idx_hbm, out_x, out_pt,
                  idx_v, posg_v, xbuf, pbuf, sgx, sgp, swx, swp):
    wid = lax.axis_index("s") * NC + lax.axis_index("c")
    graph = wid // WPG
    lbase = (wid % WPG) * RPW

    pltpu.sync_copy(idx_hbm.at[graph, pl.ds(lbase, RPW)], idx_v)
    posg_dma = pltpu.async_copy(post_hbm.at[graph], posg_v, sgp)

    gx, wx = {}, {}
    x_graph = x_hbm.at[graph]

    def start_gather(c):
        gx[c] = pltpu.async_copy(
            x_graph.at[idx_v.at[pl.ds(c * CH, CH)]], xbuf.at[c % NBUF], sgx)

    for c in range(min(NBUF, NCH)):
        start_gather(c)

    posg_dma.wait()
    for g in range(RPW // L):
        li = idx_v[pl.ds(g * L, L)]
        for j in range(PD):
            pbuf[j, pl.ds(g * L, L)] = plsc.load_gather(posg_v, [li + (j * N)])
    wpos = pltpu.async_copy(
        pbuf, out_pt.at[graph, :, pl.ds(lbase, RPW)], swp)

    for c in range(NCH):
        gx[c].wait()
        wx[c] = pltpu.async_copy(
            xbuf.at[c % NBUF], out_x.at[graph, pl.ds(lbase + c * CH, CH)], swx)
        n = c + NBUF
        if n < NCH:
            wx[c].wait()
            start_gather(n)

    for c in range(max(0, NCH - NBUF), NCH):
        wx[c].wait()
    wpos.wait()


def kernel(x, pos, mask_idx):
    pos_t = jnp.swapaxes(pos, 1, 2).reshape(B, PD * N)
    ox, opt = _node_drop_sc(x, pos_t, mask_idx.astype(jnp.int32))
    return ox, jnp.swapaxes(opt, 1, 2)


# speedup vs baseline: 1.0186x; 1.0186x over previous
import functools

import jax
import jax.numpy as jnp
from jax import lax
from jax.experimental import pallas as pl
from jax.experimental.pallas import tpu as pltpu
from jax.experimental.pallas import tpu_sc as plsc

B, N, D, K = 16, 4096, 512, 2048
PD = 3
NC, NS = 2, 16
NW = NC * NS
WPG = NW // B
RPW = (B * K) // NW
CH = 64
NBUF = 3
NCH = RPW // CH
L = 16

_mesh = plsc.VectorSubcoreMesh(core_axis_name="c", subcore_axis_name="s")


@functools.partial(
    pl.kernel,
    out_type=(
        jax.ShapeDtypeStruct((B, K, D), jnp.float32),
        jax.ShapeDtypeStruct((B, PD, K), jnp.float32),
    ),
    mesh=_mesh,
    compiler_params=pltpu.CompilerParams(needs_layout_passes=False),
    scratch_types=[
        pltpu.VMEM((RPW,), jnp.int32),
        pltpu.VMEM((PD * N,), jnp.float32),
        pltpu.VMEM((NBUF, CH, D), jnp.float32),
        pltpu.VMEM((PD, RPW), jnp.float32),
        pltpu.SemaphoreType.DMA,
        pltpu.SemaphoreType.DMA,
        pltpu.SemaphoreType.DMA,
        pltpu.SemaphoreType.DMA,
    ],
)
def _node_drop_sc(x_hbm, post_hbm, idx_hbm, out_x, out_pt,
                  idx_v, posg_v, xbuf, pbuf, sgx, sgp, swx, swp):
    wid = lax.axis_index("s") * NC + lax.axis_index("c")
    graph = wid // WPG
    lbase = (wid % WPG) * RPW

    pltpu.sync_copy(idx_hbm.at[graph, pl.ds(lbase, RPW)], idx_v)
    posg_dma = pltpu.async_copy(post_hbm.at[graph], posg_v, sgp)

    gx, wx = {}, {}
    x_graph = x_hbm.at[graph]

    def start_gather(c):
        gx[c] = pltpu.async_copy(
            x_graph.at[idx_v.at[pl.ds(c * CH, CH)]], xbuf.at[c % NBUF], sgx)

    for c in range(min(NBUF, NCH)):
        start_gather(c)

    posg_dma.wait()

    def _pos_body(g, _):
        o = g * L
        li = idx_v[pl.ds(o, L)]
        for j in range(PD):
            pbuf[j, pl.ds(o, L)] = plsc.load_gather(posg_v, [li + (j * N)])
        return _

    lax.fori_loop(0, RPW // L, _pos_body, None, unroll=4)
    wpos = pltpu.async_copy(
        pbuf, out_pt.at[graph, :, pl.ds(lbase, RPW)], swp)

    for c in range(NCH):
        gx[c].wait()
        wx[c] = pltpu.async_copy(
            xbuf.at[c % NBUF], out_x.at[graph, pl.ds(lbase + c * CH, CH)], swx)
        n = c + NBUF
        if n < NCH:
            wx[c].wait()
            start_gather(n)

    for c in range(max(0, NCH - NBUF), NCH):
        wx[c].wait()
    wpos.wait()


def kernel(x, pos, mask_idx):
    pos_t = jnp.swapaxes(pos, 1, 2).reshape(B, PD * N)
    ox, opt = _node_drop_sc(x, pos_t, mask_idx.astype(jnp.int32))
    return ox, jnp.swapaxes(opt, 1, 2)


# speedup vs baseline: 1.0435x; 1.0244x over previous
import functools

import jax
import jax.numpy as jnp
from jax import lax
from jax.experimental import pallas as pl
from jax.experimental.pallas import tpu as pltpu
from jax.experimental.pallas import tpu_sc as plsc

B, N, D, K = 16, 4096, 512, 2048
PD = 3
NC, NS = 2, 16
NW = NC * NS
WPG = NW // B
RPW = (B * K) // NW
CH = 64
NBUF = 3
NCH = RPW // CH
L = 16

_mesh = plsc.VectorSubcoreMesh(core_axis_name="c", subcore_axis_name="s")


@functools.partial(
    pl.kernel,
    out_type=(
        jax.ShapeDtypeStruct((B, K, D), jnp.float32),
        jax.ShapeDtypeStruct((B, PD, K), jnp.float32),
    ),
    mesh=_mesh,
    compiler_params=pltpu.CompilerParams(needs_layout_passes=False),
    scratch_types=[
        pltpu.VMEM((RPW,), jnp.int32),
        pltpu.VMEM((PD * N,), jnp.float32),
        pltpu.VMEM((NBUF, CH, D), jnp.float32),
        pltpu.VMEM((PD, RPW), jnp.float32),
        pltpu.SemaphoreType.DMA,
        pltpu.SemaphoreType.DMA,
        pltpu.SemaphoreType.DMA,
        pltpu.SemaphoreType.DMA,
    ],
)
def _node_drop_sc(x_hbm, post_hbm, idx_hbm, out_x, out_pt,
                  idx_v, posg_v, xbuf, pbuf, sgx, sgp, swx, swp):
    wid = lax.axis_index("s") * NC + lax.axis_index("c")
    graph = wid // WPG
    lbase = (wid % WPG) * RPW

    pltpu.sync_copy(idx_hbm.at[graph, pl.ds(lbase, RPW)], idx_v)
    posg_dma = pltpu.async_copy(post_hbm.at[graph], posg_v, sgp)

    x_graph = x_hbm.at[graph]

    for c in range(min(NBUF, NCH)):
        pltpu.async_copy(
            x_graph.at[idx_v.at[pl.ds(c * CH, CH)]], xbuf.at[c], sgx)

    posg_dma.wait()

    def _pos_body(g, _):
        o = g * L
        li = idx_v[pl.ds(o, L)]
        for j in range(PD):
            pbuf[j, pl.ds(o, L)] = plsc.load_gather(posg_v, [li + (j * N)])
        return _

    lax.fori_loop(0, RPW // L, _pos_body, None, unroll=4)
    wpos = pltpu.async_copy(
        pbuf, out_pt.at[graph, :, pl.ds(lbase, RPW)], swp)

    def _drain(sem):
        pltpu.make_async_copy(x_graph.at[pl.ds(0, CH)], xbuf.at[0], sem).wait()

    def _chunk_body(c, _):
        _drain(sgx)
        slot = lax.rem(c, NBUF)
        pltpu.async_copy(
            xbuf.at[slot], out_x.at[graph, pl.ds(lbase + c * CH, CH)], swx)

        @pl.when(c + NBUF < NCH)
        def _():
            _drain(swx)
            pltpu.async_copy(
                x_graph.at[idx_v.at[pl.ds((c + NBUF) * CH, CH)]],
                xbuf.at[slot], sgx)

        return _

    lax.fori_loop(0, NCH, _chunk_body, None)
    for _ in range(min(NBUF, NCH)):
        _drain(swx)
    wpos.wait()


def kernel(x, pos, mask_idx):
    pos_t = jnp.swapaxes(pos, 1, 2).reshape(B, PD * N)
    ox, opt = _node_drop_sc(x, pos_t, mask_idx.astype(jnp.int32))
    return ox, jnp.swapaxes(opt, 1, 2)


# speedup vs baseline: 1.0486x; 1.0049x over previous
import functools

import jax
import jax.numpy as jnp
from jax import lax
from jax.experimental import pallas as pl
from jax.experimental.pallas import tpu as pltpu
from jax.experimental.pallas import tpu_sc as plsc

B, N, D, K = 16, 4096, 512, 2048
PD = 3
NC, NS = 2, 16
NW = NC * NS
WPG = NW // B
RPW = (B * K) // NW
CH = 64
NBUF = 3
NCH = RPW // CH
L = 16

_mesh = plsc.VectorSubcoreMesh(core_axis_name="c", subcore_axis_name="s")


@functools.partial(
    pl.kernel,
    out_type=(
        jax.ShapeDtypeStruct((B, K, D), jnp.float32),
        jax.ShapeDtypeStruct((B, PD, K), jnp.float32),
    ),
    mesh=_mesh,
    compiler_params=pltpu.CompilerParams(needs_layout_passes=False),
    scratch_types=[
        pltpu.VMEM((RPW,), jnp.int32),
        pltpu.VMEM((PD * N,), jnp.float32),
        pltpu.VMEM((NBUF, CH, D), jnp.float32),
        pltpu.VMEM((PD, RPW), jnp.float32),
        pltpu.SemaphoreType.DMA,
        pltpu.SemaphoreType.DMA,
        pltpu.SemaphoreType.DMA,
        pltpu.SemaphoreType.DMA,
    ],
)
def _node_drop_sc(x_hbm, post_hbm, idx_hbm, out_x, out_pt,
                  idx_v, posg_v, xbuf, pbuf, sgx, sgp, swx, swp):
    wid = lax.axis_index("s") * NC + lax.axis_index("c")
    graph = wid // WPG
    lbase = (wid % WPG) * RPW

    pltpu.sync_copy(idx_hbm.at[graph, pl.ds(lbase, RPW)], idx_v)
    posg_dma = pltpu.async_copy(post_hbm.at[graph], posg_v, sgp)

    x_graph = x_hbm.at[graph]

    def _prime_body(c, _):
        pltpu.async_copy(
            x_graph.at[idx_v.at[pl.ds(c * CH, CH)]], xbuf.at[c], sgx)
        return _

    lax.fori_loop(0, min(NBUF, NCH), _prime_body, None)

    posg_dma.wait()

    def _pos_body(g, _):
        o = g * L
        li = idx_v[pl.ds(o, L)]
        for j in range(PD):
            pbuf[j, pl.ds(o, L)] = plsc.load_gather(posg_v, [li + (j * N)])
        return _

    lax.fori_loop(0, RPW // L, _pos_body, None, unroll=2)
    wpos = pltpu.async_copy(
        pbuf, out_pt.at[graph, :, pl.ds(lbase, RPW)], swp)

    def _drain(sem):
        pltpu.make_async_copy(x_graph.at[pl.ds(0, CH)], xbuf.at[0], sem).wait()

    def _chunk_body(c, _):
        _drain(sgx)
        slot = lax.rem(c, NBUF)
        pltpu.async_copy(
            xbuf.at[slot], out_x.at[graph, pl.ds(lbase + c * CH, CH)], swx)

        @pl.when(c + NBUF < NCH)
        def _():
            _drain(swx)
            pltpu.async_copy(
                x_graph.at[idx_v.at[pl.ds((c + NBUF) * CH, CH)]],
                xbuf.at[slot], sgx)

        return _

    lax.fori_loop(0, NCH, _chunk_body, None)
    for _ in range(min(NBUF, NCH)):
        _drain(swx)
    wpos.wait()


def kernel(x, pos, mask_idx):
    pos_t = jnp.swapaxes(pos, 1, 2).reshape(B, PD * N)
    ox, opt = _node_drop_sc(x, pos_t, mask_idx.astype(jnp.int32))
    return ox, jnp.swapaxes(opt, 1, 2)
